# baseline (speedup 1.0000x reference)
import jax
import jax.numpy as jnp
from jax.experimental import pallas as pl
from jax.experimental.pallas import tpu as pltpu

BATCH = 16384
DIM = 64
ROWS_PER_STEP = 8
GRID = BATCH // ROWS_PER_STEP


def _body(*refs):
    head_s, rel_s, tail_s = refs[:3]
    h_refs = refs[3:3 + ROWS_PER_STEP]
    t_refs = refs[3 + ROWS_PER_STEP:3 + 2 * ROWS_PER_STEP]
    rtab_ref = refs[3 + 2 * ROWS_PER_STEP]
    out_ref = refs[4 + 2 * ROWS_PER_STEP]
    i = pl.program_id(0)
    for k in range(ROWS_PER_STEP):
        b = i * ROWS_PER_STEP + k
        hrow = head_s[b] % 8
        trow = tail_s[b] % 8
        rrow = rel_s[b]
        out_ref[k, :] = (
            h_refs[k][hrow, :] + rtab_ref[rrow, :] - t_refs[k][trow, :]
        )


def _transe_tc(head, relation, tail, entity_emb, relation_emb):
    def h_map(k):
        return lambda i, h, r, t: (h[ROWS_PER_STEP * i + k] // 8, 0)

    def t_map(k):
        return lambda i, h, r, t: (t[ROWS_PER_STEP * i + k] // 8, 0)

    tile_spec = lambda m: pl.BlockSpec((8, DIM), m)
    in_specs = (
        [tile_spec(h_map(k)) for k in range(ROWS_PER_STEP)]
        + [tile_spec(t_map(k)) for k in range(ROWS_PER_STEP)]
        + [pl.BlockSpec((3, DIM), lambda i, h, r, t: (0, 0))]
    )
    grid_spec = pltpu.PrefetchScalarGridSpec(
        num_scalar_prefetch=3,
        grid=(GRID,),
        in_specs=in_specs,
        out_specs=pl.BlockSpec((ROWS_PER_STEP, DIM), lambda i, h, r, t: (i, 0)),
    )
    return pl.pallas_call(
        _body,
        grid_spec=grid_spec,
        out_shape=jax.ShapeDtypeStruct((BATCH, DIM), jnp.float32),
        compiler_params=pltpu.CompilerParams(
            dimension_semantics=("parallel",),
        ),
    )(
        head, relation, tail,
        *([entity_emb] * ROWS_PER_STEP),
        *([entity_emb] * ROWS_PER_STEP),
        relation_emb,
    )


@jax.jit
def kernel(head, relation, tail, entity_emb, relation_emb):
    head = head.astype(jnp.int32)
    relation = relation.astype(jnp.int32)
    tail = tail.astype(jnp.int32)
    return _transe_tc(head, relation, tail, entity_emb, relation_emb)


# speedup vs baseline: 2.5700x; 2.5700x over previous
import dataclasses
import functools

import jax
import jax.numpy as jnp
from jax import lax
from jax.experimental import pallas as pl
from jax.experimental.pallas import tpu as pltpu
from jax.experimental.pallas import tpu_sc as plsc

BATCH = 16384
DIM = 64
LANES = 16
NUM_CORES = 2
NUM_SUBCORES = 16
NUM_WORKERS = NUM_CORES * NUM_SUBCORES
B_PER_W = BATCH // NUM_WORKERS
CHUNK = 128


def _transe_sc(hslice, hoff, rel, tslice, toff, ent2, reltab):
    mesh = plsc.VectorSubcoreMesh(core_axis_name="c", subcore_axis_name="s")
    cp = pltpu.CompilerParams()
    if "needs_layout_passes" in pltpu.CompilerParams.__dataclass_fields__:
        cp = dataclasses.replace(cp, needs_layout_passes=False)

    @functools.partial(
        pl.kernel,
        mesh=mesh,
        compiler_params=cp,
        out_type=jax.ShapeDtypeStruct((BATCH, DIM), jnp.float32),
        scratch_types=[
            pltpu.VMEM((B_PER_W,), jnp.int32),
            pltpu.VMEM((B_PER_W,), jnp.int32),
            pltpu.VMEM((B_PER_W,), jnp.int32),
            pltpu.VMEM((B_PER_W,), jnp.int32),
            pltpu.VMEM((B_PER_W,), jnp.int32),
            pltpu.VMEM((CHUNK, 2 * DIM), jnp.float32),
            pltpu.VMEM((CHUNK, 2 * DIM), jnp.float32),
            pltpu.VMEM((3, DIM), jnp.float32),
            pltpu.VMEM((CHUNK, DIM), jnp.float32),
            pltpu.SemaphoreType.DMA,
            pltpu.SemaphoreType.DMA,
        ],
    )
    def k(hsl_hbm, hof_hbm, rel_hbm, tsl_hbm, tof_hbm, ent2_hbm,
          reltab_hbm, out_hbm,
          hsl_v, hof_v, rel_v, tsl_v, tof_v, h2_v, t2_v, rtab_v, out_v,
          hsem, tsem):
        wid = lax.axis_index("s") * NUM_CORES + lax.axis_index("c")
        base = wid * B_PER_W
        pltpu.sync_copy(hsl_hbm.at[pl.ds(base, B_PER_W)], hsl_v)
        pltpu.sync_copy(hof_hbm.at[pl.ds(base, B_PER_W)], hof_v)
        pltpu.sync_copy(rel_hbm.at[pl.ds(base, B_PER_W)], rel_v)
        pltpu.sync_copy(tsl_hbm.at[pl.ds(base, B_PER_W)], tsl_v)
        pltpu.sync_copy(tof_hbm.at[pl.ds(base, B_PER_W)], tof_v)
        pltpu.sync_copy(reltab_hbm, rtab_v)

        lane = lax.broadcasted_iota(jnp.int32, (LANES,), 0)

        @pl.loop(0, B_PER_W, step=CHUNK)
        def _(c):
            cp_h = pltpu.make_async_copy(
                ent2_hbm.at[hsl_v.at[pl.ds(c, CHUNK)]], h2_v, hsem
            )
            cp_t = pltpu.make_async_copy(
                ent2_hbm.at[tsl_v.at[pl.ds(c, CHUNK)]], t2_v, tsem
            )
            cp_h.start()
            cp_t.start()
            cp_h.wait()
            cp_t.wait()

            @pl.loop(0, CHUNK)
            def _(i):
                g = jnp.full((LANES,), c + i, jnp.int32)
                iv = jnp.full((LANES,), i, jnp.int32)
                ho = plsc.load_gather(hof_v, [g])
                to = plsc.load_gather(tof_v, [g])
                rv = plsc.load_gather(rel_v, [g])
                for j in range(DIM // LANES):
                    ln = lane + (j * LANES)
                    hc = plsc.load_gather(h2_v, [iv, ho + ln])
                    tc = plsc.load_gather(t2_v, [iv, to + ln])
                    rc = plsc.load_gather(rtab_v, [rv, ln])
                    out_v.at[i, pl.ds(j * LANES, LANES)][...] = hc + rc - tc

            pltpu.sync_copy(out_v, out_hbm.at[pl.ds(base + c, CHUNK)])

    return k(hslice, hoff, rel, tslice, toff, ent2, reltab)


@jax.jit
def kernel(head, relation, tail, entity_emb, relation_emb):
    head = head.astype(jnp.int32)
    relation = relation.astype(jnp.int32)
    tail = tail.astype(jnp.int32)
    ent2 = jnp.reshape(entity_emb, (entity_emb.shape[0] // 2, 2 * DIM))
    return _transe_sc(
        head >> 1, (head & 1) * DIM, relation,
        tail >> 1, (tail & 1) * DIM,
        ent2, relation_emb,
    )
